# Initial kernel scaffold; baseline (speedup 1.0000x reference)
import jax
import jax.numpy as jnp
from jax.experimental import pallas as pl


def kernel(x, edge_index, Wl0, bl0, Wr0, Wl1, bl1, Wr1, Wl2, bl2, Wr2, Wo, bo):
    raise NotImplementedError("write your pallas kernel here")


# speedup vs baseline: 6.1210x; 6.1210x over previous
import functools

import jax
import jax.numpy as jnp
from jax import lax
from jax.experimental import pallas as pl
from jax.experimental.pallas import tpu as pltpu
from jax.experimental.pallas import tpu_sc as plsc

N = 10000
E = 320000
D = 128
OUT = 2

NC = 2
NS = 16
NW = NC * NS

B = 80
CE = E // NW
K = CE // B
NP = 10240
RPT = NP // NS
CPT = NP // NS


_MESH = plsc.VectorSubcoreMesh(core_axis_name="c", subcore_axis_name="s")


def _seg_body(h_hbm, src2, dst2, zrows, p_out, idx_s, idx_d, rows, acc, sem):
  cid = lax.axis_index("c")
  sid = lax.axis_index("s")
  wid = cid * NS + sid

  pltpu.sync_copy(src2.at[wid], idx_s)
  pltpu.sync_copy(dst2.at[wid], idx_d)
  pltpu.sync_copy(zrows, acc.at[pl.ds(sid * RPT, RPT), :])
  plsc.subcore_barrier()

  def chunk(c, _):
    pltpu.async_copy(h_hbm.at[idx_s.at[c]], rows, sem).wait()
    pltpu.sync_copy(rows, acc.at[idx_d.at[c]], add=True)
    return _

  lax.fori_loop(0, K, chunk, None)
  plsc.subcore_barrier()

  pltpu.sync_copy(acc.at[pl.ds(sid * RPT, RPT), :],
                  p_out.at[cid, pl.ds(sid * RPT, RPT), :])


_sc_seg = pl.kernel(
    _seg_body,
    out_type=(jax.ShapeDtypeStruct((NC, NP, D), jnp.float32),),
    mesh=_MESH,
    scratch_types=[
        pltpu.VMEM((K, B), jnp.int32),
        pltpu.VMEM((K, B), jnp.int32),
        pltpu.VMEM((B, D), jnp.float32),
        pltpu.VMEM_SHARED((NP, D), jnp.float32),
        pltpu.SemaphoreType.DMA,
    ])


def _cnt_body(dst2, zrows, ones_hbm, c_out, idx_d, ones_v, cacc):
  cid = lax.axis_index("c")
  sid = lax.axis_index("s")
  wid = cid * NS + sid

  pltpu.sync_copy(dst2.at[wid], idx_d)
  pltpu.sync_copy(zrows, cacc.at[pl.ds(sid * RPT, RPT), :])
  pltpu.sync_copy(ones_hbm, ones_v)
  plsc.subcore_barrier()

  def chunk(c, _):
    pltpu.sync_copy(ones_v, cacc.at[idx_d.at[c]], add=True)
    return _

  lax.fori_loop(0, K, chunk, None)
  plsc.subcore_barrier()

  pltpu.sync_copy(cacc.at[pl.ds(sid * RPT, RPT), :],
                  c_out.at[cid, pl.ds(sid * RPT, RPT), :])


_sc_counts = pl.kernel(
    _cnt_body,
    out_type=(jax.ShapeDtypeStruct((NC, NP, D), jnp.float32),),
    mesh=_MESH,
    scratch_types=[
        pltpu.VMEM((K, B), jnp.int32),
        pltpu.VMEM((B, D), jnp.float32),
        pltpu.VMEM_SHARED((NP, D), jnp.float32),
    ])

R = 1024


def _dense_body(p_ref, c_ref, h_ref, wl_ref, bl_ref, wr_ref, o_ref, *, relu):
  cnt = c_ref[0][:, 0:1] + c_ref[1][:, 0:1]
  inv = 1.0 / jnp.maximum(cnt, 1.0)
  agg = (p_ref[0] + p_ref[1]) * inv
  dn = (((1,), (1,)), ((), ()))
  acc = lax.dot_general(agg, wl_ref[...], dn,
                        precision=lax.Precision.HIGHEST,
                        preferred_element_type=jnp.float32)
  acc += lax.dot_general(h_ref[...], wr_ref[...], dn,
                         precision=lax.Precision.HIGHEST,
                         preferred_element_type=jnp.float32)
  acc += bl_ref[...]
  o_ref[...] = jnp.maximum(acc, 0.0) if relu else acc


def _final_body(p_ref, c_ref, h_ref, wl_ref, bl_ref, wr_ref, wo_ref, bo_ref,
                o_ref):
  cnt = c_ref[0][:, 0:1] + c_ref[1][:, 0:1]
  inv = 1.0 / jnp.maximum(cnt, 1.0)
  agg = (p_ref[0] + p_ref[1]) * inv
  dn = (((1,), (1,)), ((), ()))
  acc = lax.dot_general(agg, wl_ref[...], dn,
                        precision=lax.Precision.HIGHEST,
                        preferred_element_type=jnp.float32)
  acc += lax.dot_general(h_ref[...], wr_ref[...], dn,
                         precision=lax.Precision.HIGHEST,
                         preferred_element_type=jnp.float32)
  acc += bl_ref[...]
  o_ref[...] = lax.dot_general(acc, wo_ref[...], dn,
                               precision=lax.Precision.HIGHEST,
                               preferred_element_type=jnp.float32) + bo_ref[...]


_W_SPEC = pl.BlockSpec((D, D), lambda i: (0, 0))
_B_SPEC = pl.BlockSpec((1, D), lambda i: (0, 0))
_P_SPEC = pl.BlockSpec((NC, R, D), lambda i: (0, i, 0))
_C_SPEC = pl.BlockSpec((NC, R, D), lambda i: (0, i, 0))
_H_SPEC = pl.BlockSpec((R, D), lambda i: (i, 0))


def _dense_layer(P, C, h, Wl, bl, Wr, relu):
  return pl.pallas_call(
      functools.partial(_dense_body, relu=relu),
      grid=(NP // R,),
      in_specs=[_P_SPEC, _C_SPEC, _H_SPEC, _W_SPEC, _B_SPEC, _W_SPEC],
      out_specs=_H_SPEC,
      out_shape=jax.ShapeDtypeStruct((NP, D), jnp.float32),
  )(P, C, h, Wl, bl, Wr)


def _final_layer(P, C, h, Wl, bl, Wr, Wo, bo):
  return pl.pallas_call(
      _final_body,
      grid=(NP // R,),
      in_specs=[_P_SPEC, _C_SPEC, _H_SPEC, _W_SPEC, _B_SPEC, _W_SPEC,
                pl.BlockSpec((OUT, D), lambda i: (0, 0)),
                pl.BlockSpec((1, OUT), lambda i: (0, 0))],
      out_specs=pl.BlockSpec((R, OUT), lambda i: (i, 0)),
      out_shape=jax.ShapeDtypeStruct((NP, OUT), jnp.float32),
  )(P, C, h, Wl, bl, Wr, Wo, bo)


@jax.jit
def kernel(x, edge_index, Wl0, bl0, Wr0, Wl1, bl1, Wr1, Wl2, bl2, Wr2, Wo, bo):
  src2 = edge_index[0].reshape(NW, K, B)
  dst2 = edge_index[1].reshape(NW, K, B)
  zrows = jnp.zeros((RPT, D), jnp.float32)
  ones = jnp.ones((B, D), jnp.float32)
  bl0r, bl1r, bl2r = bl0.reshape(1, D), bl1.reshape(1, D), bl2.reshape(1, D)
  bor = bo.reshape(1, OUT)
  xp = jnp.concatenate([x, jnp.zeros((NP - N, D), jnp.float32)], axis=0)

  (C,) = _sc_counts(dst2, zrows, ones)
  (P0,) = _sc_seg(xp, src2, dst2, zrows)
  h1 = _dense_layer(P0, C, xp, Wl0, bl0r, Wr0, True)
  (P1,) = _sc_seg(h1, src2, dst2, zrows)
  h2 = _dense_layer(P1, C, h1, Wl1, bl1r, Wr1, True)
  (P2,) = _sc_seg(h2, src2, dst2, zrows)
  return _final_layer(P2, C, h2, Wl2, bl2r, Wr2, Wo, bor)[:N]
